# baseline (speedup 1.0000x reference)
import functools

import jax
import jax.numpy as jnp
from jax import lax
from jax.experimental import pallas as pl
from jax.experimental.pallas import tpu as pltpu
from jax.experimental.pallas import tpu_sc as plsc

NC = 2
NS = 16
L = 16
C = 128
DH = 64
NB = 4


def _pre_body(x_ref, w_ref, att2_ref, hp_ref, al_ref, mx_ref):
    i = pl.program_id(0)
    h = jnp.dot(x_ref[...], w_ref[...], preferred_element_type=jnp.float32)
    ab = jnp.dot(h, att2_ref[...], preferred_element_type=jnp.float32)
    hp_ref[0, :, :] = h[:, :DH]
    hp_ref[1, :, :] = h[:, DH:]
    al_ref[...] = ab
    bm = jnp.max(ab, axis=0, keepdims=True)

    @pl.when(i == 0)
    def _():
        mx_ref[...] = bm

    @pl.when(i != 0)
    def _():
        mx_ref[...] = jnp.maximum(mx_ref[...], bm)


def _post_body(acc_ref, den_ref, bias_ref, wp_ref, bp_ref, out_ref):
    den = jnp.sum(den_ref[...], axis=1, keepdims=True) + 1e-16
    g0 = jnp.maximum(acc_ref[0] / den + bias_ref[:, :DH], 0.0)
    g1 = jnp.maximum(acc_ref[1] / den + bias_ref[:, DH:], 0.0)
    o = (jnp.dot(g0, wp_ref[:DH, :], preferred_element_type=jnp.float32)
         + jnp.dot(g1, wp_ref[DH:, :], preferred_element_type=jnp.float32)
         + bp_ref[...])
    out_ref[...] = jnp.maximum(o, 0.0)


def _edge_body(hp_hbm, edge_hbm, asrc_hbm, adst_hbm, shift_hbm,
               acc_hbm, den_hbm,
               asrc_v, adst_v, eidx0, eidx1, eidx2, eidx3,
               rows0, rows1, rows2, rows3, denr0, denr1, denr2, denr3,
               sdix0, sdix1, sdix2, sdix3,
               shift_v, acc_sh, den_sh, isem, rsem, ssem, dsem,
               n_nodes, n_edges):
    c = lax.axis_index("c")
    s = lax.axis_index("s")
    eidx = (eidx0, eidx1, eidx2, eidx3)
    rows = (rows0, rows1, rows2, rows3)
    denr = (denr0, denr1, denr2, denr3)
    sdix = (sdix0, sdix1, sdix2, sdix3)
    rows_per_tile = n_nodes // NS
    zr = 125
    nz = rows_per_tile // zr

    pltpu.sync_copy(asrc_hbm, asrc_v)
    pltpu.sync_copy(adst_hbm, adst_v)
    pltpu.sync_copy(shift_hbm, shift_v)

    zeros16 = jnp.zeros((L,), jnp.float32)

    def zrow(r, carry):
        for j in range(DH // L):
            rows0[r, pl.ds(j * L, L)] = zeros16
        for q in range(NB):
            denr[q][r, :] = zeros16
        return carry

    lax.fori_loop(0, C, zrow, 0)
    base = s * rows_per_tile
    for q in range(nz):
        pltpu.sync_copy(rows0.at[pl.ds(0, zr)], acc_sh.at[pl.ds(base + q * zr, zr)])

    @pl.when(c == 0)
    def _():
        for q in range(nz):
            pltpu.sync_copy(denr0.at[pl.ds(0, zr)],
                            den_sh.at[pl.ds(base + q * zr, zr)])

    plsc.subcore_barrier()

    shift = shift_v[...]
    iota16 = lax.iota(jnp.int32, L)
    zeros16i = jnp.zeros((L,), jnp.int32)
    nchunks = n_edges // C
    full = nchunks // NS
    rem = nchunks - full * NS
    nct = full + jnp.where(s < rem, 1, 0)

    def cid_of(u):
        return jnp.where(u < full, s + NS * u, NS * full + s)

    def issue_idx(u, q):
        e0 = cid_of(u) * C
        pltpu.async_copy(edge_hbm.at[:, pl.ds(e0, C)], eidx[q], isem.at[q])

    def wait_idx(u, q):
        e0 = cid_of(u) * C
        pltpu.make_async_copy(edge_hbm.at[:, pl.ds(e0, C)], eidx[q], isem.at[q]).wait()

    def issue_rows(q):
        pltpu.async_copy(hp_hbm.at[c].at[eidx[q].at[0]], rows[q], rsem.at[q])

    def wait_rows(q):
        pltpu.make_async_copy(hp_hbm.at[c].at[eidx[q].at[0]], rows[q], rsem.at[q]).wait()

    def compute(q):
        def grp_body(g, carry):
            si = eidx[q][0, pl.ds(g * L, L)]
            di = eidx[q][1, pl.ds(g * L, L)]
            sdix[q][pl.ds(g * L, L)] = di
            a = plsc.load_gather(asrc_v, [si]) + plsc.load_gather(adst_v, [di])
            a = jnp.where(a >= 0, a, 0.2 * a)
            p = jnp.exp(a - shift)

            @pl.when(c == 0)
            def _():
                plsc.store_scatter(denr[q], [g * L + iota16, zeros16i], p)

            for k in range(L):
                ps = p[k]
                r = g * L + k
                for j in range(DH // L):
                    rows[q][r, pl.ds(j * L, L)] = rows[q][r, pl.ds(j * L, L)] * ps
            return carry

        lax.fori_loop(0, C // L, grp_body, 0)

    def issue_scatter(q):
        pltpu.async_copy(rows[q], acc_sh.at[sdix[q]], ssem.at[q], add=True)

        @pl.when(c == 0)
        def _():
            pltpu.async_copy(denr[q], den_sh.at[sdix[q]], dsem.at[q], add=True)

    def wait_scatter(q):
        pltpu.make_async_copy(rows[q], acc_sh.at[sdix[q]], ssem.at[q]).wait()

        @pl.when(c == 0)
        def _():
            pltpu.make_async_copy(denr[q], den_sh.at[sdix[q]], dsem.at[q]).wait()

    issue_idx(0, 0)
    issue_idx(1, 1)
    issue_idx(2, 2)
    wait_idx(0, 0)
    issue_rows(0)
    wait_idx(1, 1)
    issue_rows(1)

    def step(v, q):
        q1 = (q + 1) % NB
        q2 = (q + 2) % NB
        q3 = (q + 3) % NB

        @pl.when((v >= 2) & (v - 2 < nct))
        def _():
            wait_scatter(q2)

        @pl.when(v + 3 < nct)
        def _():
            issue_idx(v + 3, q3)

        @pl.when(v + 2 < nct)
        def _():
            wait_idx(v + 2, q2)
            issue_rows(q2)

        @pl.when(v < nct)
        def _():
            wait_rows(q)
            compute(q)
            issue_scatter(q)

    def quad(i, carry):
        for k in range(NB):
            step(NB * i + k, k)
        return carry

    niter = (full + 1 + 2 + NB - 1) // NB
    lax.fori_loop(0, niter, quad, 0)

    plsc.subcore_barrier()
    for q in range(nz):
        sl = pl.ds(base + q * zr, zr)
        pltpu.sync_copy(acc_sh.at[sl], acc_hbm.at[c, sl])

    @pl.when(c == 0)
    def _():
        for q in range(nz):
            sl = pl.ds(base + q * zr, zr)
            pltpu.sync_copy(den_sh.at[sl], den_hbm.at[sl])


def kernel(x, edge_index, W_gat, att_src, att_dst, bias_gat, W_proj, b_proj):
    n, d = x.shape
    e = edge_index.shape[1]
    bn = 1000
    grid = n // bn

    att2 = jnp.stack([att_src, att_dst], axis=1)

    hp, alphas, mx = pl.pallas_call(
        _pre_body,
        grid=(grid,),
        in_specs=[
            pl.BlockSpec((bn, d), lambda i: (i, 0)),
            pl.BlockSpec((d, d), lambda i: (0, 0)),
            pl.BlockSpec((d, 2), lambda i: (0, 0)),
        ],
        out_specs=[
            pl.BlockSpec((NC, bn, DH), lambda i: (0, i, 0)),
            pl.BlockSpec((bn, 2), lambda i: (i, 0)),
            pl.BlockSpec((1, 2), lambda i: (0, 0)),
        ],
        out_shape=[
            jax.ShapeDtypeStruct((NC, n, DH), jnp.float32),
            jax.ShapeDtypeStruct((n, 2), jnp.float32),
            jax.ShapeDtypeStruct((1, 2), jnp.float32),
        ],
    )(x, W_gat, att2)

    a_src = alphas[:, 0]
    a_dst = alphas[:, 1]
    shift = jnp.full((L,), jnp.maximum(mx[0, 0] + mx[0, 1], 0.0), jnp.float32)

    mesh = plsc.VectorSubcoreMesh(
        core_axis_name="c", subcore_axis_name="s", num_cores=NC, num_subcores=NS
    )
    edge_kernel = functools.partial(
        pl.kernel,
        out_type=(
            jax.ShapeDtypeStruct((NC, n, DH), jnp.float32),
            jax.ShapeDtypeStruct((n, L), jnp.float32),
        ),
        mesh=mesh,
        compiler_params=pltpu.CompilerParams(
            use_tc_tiling_on_sc=False, needs_layout_passes=False
        ),
        scratch_types=[
            pltpu.VMEM((n,), jnp.float32),
            pltpu.VMEM((n,), jnp.float32),
        ] + [pltpu.VMEM((2, C), jnp.int32) for _ in range(NB)]
          + [pltpu.VMEM((C, DH), jnp.float32) for _ in range(NB)]
          + [pltpu.VMEM((C, L), jnp.float32) for _ in range(NB)]
          + [pltpu.VMEM((C,), jnp.int32) for _ in range(NB)]
          + [
            pltpu.VMEM((L,), jnp.float32),
            pltpu.VMEM_SHARED((n, DH), jnp.float32),
            pltpu.VMEM_SHARED((n, L), jnp.float32),
            pltpu.SemaphoreType.DMA((NB,)),
            pltpu.SemaphoreType.DMA((NB,)),
            pltpu.SemaphoreType.DMA((NB,)),
            pltpu.SemaphoreType.DMA((NB,)),
        ],
    )(functools.partial(_edge_body, n_nodes=n, n_edges=e))

    acc, den = edge_kernel(hp, edge_index, a_src, a_dst, shift)

    out = pl.pallas_call(
        _post_body,
        grid=(grid,),
        in_specs=[
            pl.BlockSpec((NC, bn, DH), lambda i: (0, i, 0)),
            pl.BlockSpec((bn, L), lambda i: (i, 0)),
            pl.BlockSpec((1, d), lambda i: (0, 0)),
            pl.BlockSpec((d, d), lambda i: (0, 0)),
            pl.BlockSpec((1, d), lambda i: (0, 0)),
        ],
        out_specs=pl.BlockSpec((bn, d), lambda i: (i, 0)),
        out_shape=jax.ShapeDtypeStruct((n, d), jnp.float32),
    )(acc, den, bias_gat[None, :], W_proj, b_proj[None, :])

    return out


# speedup vs baseline: 2.0477x; 2.0477x over previous
import functools

import jax
import jax.numpy as jnp
from jax import lax
from jax.experimental import pallas as pl
from jax.experimental.pallas import tpu as pltpu
from jax.experimental.pallas import tpu_sc as plsc

NC = 2
NS = 16
L = 16
C = 128
DH = 64
NB = 4


def _pre_body(x_ref, w_ref, att2_ref, hp_ref, al_ref, mx_ref):
    i = pl.program_id(0)
    h = jnp.dot(x_ref[...], w_ref[...], preferred_element_type=jnp.float32)
    ab = jnp.dot(h, att2_ref[...], preferred_element_type=jnp.float32)
    hp_ref[0, :, :] = h[:, :DH]
    hp_ref[1, :, :] = h[:, DH:]
    al_ref[...] = ab
    bm = jnp.max(ab, axis=0, keepdims=True)

    @pl.when(i == 0)
    def _():
        mx_ref[...] = bm

    @pl.when(i != 0)
    def _():
        mx_ref[...] = jnp.maximum(mx_ref[...], bm)


def _post_body(acc_ref, den_ref, bias_ref, wp_ref, bp_ref, out_ref):
    den = jnp.sum(den_ref[...], axis=1, keepdims=True) + 1e-16
    g0 = jnp.maximum(acc_ref[0] / den + bias_ref[:, :DH], 0.0)
    g1 = jnp.maximum(acc_ref[1] / den + bias_ref[:, DH:], 0.0)
    o = (jnp.dot(g0, wp_ref[:DH, :], preferred_element_type=jnp.float32)
         + jnp.dot(g1, wp_ref[DH:, :], preferred_element_type=jnp.float32)
         + bp_ref[...])
    out_ref[...] = jnp.maximum(o, 0.0)


def _edge_body(hp_hbm, edge_hbm, asrc_hbm, adst_hbm, shift_hbm,
               acc_hbm, den_hbm,
               asrc_v, adst_v, eidx0, eidx1, eidx2, eidx3,
               rows0, rows1, rows2, rows3, denr0, denr1, denr2, denr3,
               sdix0, sdix1, sdix2, sdix3,
               pbuf_v, shift_v, acc_sh, den_sh, isem, rsem, ssem, dsem,
               n_nodes, n_edges):
    c = lax.axis_index("c")
    s = lax.axis_index("s")
    eidx = (eidx0, eidx1, eidx2, eidx3)
    rows = (rows0, rows1, rows2, rows3)
    denr = (denr0, denr1, denr2, denr3)
    sdix = (sdix0, sdix1, sdix2, sdix3)
    rows_per_tile = n_nodes // NS
    zr = 125
    nz = rows_per_tile // zr

    pltpu.sync_copy(asrc_hbm, asrc_v)
    pltpu.sync_copy(adst_hbm, adst_v)
    pltpu.sync_copy(shift_hbm, shift_v)

    zeros16 = jnp.zeros((L,), jnp.float32)

    def zrow(r, carry):
        for j in range(DH // L):
            rows0[r, pl.ds(j * L, L)] = zeros16
        for q in range(NB):
            denr[q][r, :] = zeros16
        return carry

    lax.fori_loop(0, C, zrow, 0)
    base = s * rows_per_tile
    for q in range(nz):
        pltpu.sync_copy(rows0.at[pl.ds(0, zr)], acc_sh.at[pl.ds(base + q * zr, zr)])

    @pl.when(c == 0)
    def _():
        for q in range(nz):
            pltpu.sync_copy(denr0.at[pl.ds(0, zr)],
                            den_sh.at[pl.ds(base + q * zr, zr)])

    plsc.subcore_barrier()

    shift = shift_v[...]
    iota16 = lax.iota(jnp.int32, L)
    zeros16i = jnp.zeros((L,), jnp.int32)
    nchunks = n_edges // C
    full = nchunks // NS
    rem = nchunks - full * NS
    nct = full + jnp.where(s < rem, 1, 0)

    def cid_of(u):
        return jnp.where(u < full, s + NS * u, NS * full + s)

    def issue_idx(u, q):
        e0 = cid_of(u) * C
        pltpu.async_copy(edge_hbm.at[:, pl.ds(e0, C)], eidx[q], isem.at[q])

    def wait_idx(u, q):
        e0 = cid_of(u) * C
        pltpu.make_async_copy(edge_hbm.at[:, pl.ds(e0, C)], eidx[q], isem.at[q]).wait()

    def issue_rows(q):
        pltpu.async_copy(hp_hbm.at[c].at[eidx[q].at[0]], rows[q], rsem.at[q])

    def wait_rows(q):
        pltpu.make_async_copy(hp_hbm.at[c].at[eidx[q].at[0]], rows[q], rsem.at[q]).wait()

    def compute(q):
        def grp_body(g, carry):
            si = eidx[q][0, pl.ds(g * L, L)]
            di = eidx[q][1, pl.ds(g * L, L)]
            sdix[q][pl.ds(g * L, L)] = di
            a = plsc.load_gather(asrc_v, [si]) + plsc.load_gather(adst_v, [di])
            a = jnp.where(a >= 0, a, 0.2 * a)
            p = jnp.exp(a - shift)
            pbuf_v[pl.ds(g * L, L)] = p

            @pl.when(c == 0)
            def _():
                plsc.store_scatter(denr[q], [g * L + iota16, zeros16i], p)

            return carry

        lax.fori_loop(0, C // L, grp_body, 0)

        @plsc.parallel_loop(0, C, 1, unroll=8)
        def _(r):
            ps = pbuf_v[pl.ds(r, L)][0]
            for j in range(DH // L):
                rows[q][r, pl.ds(j * L, L)] = rows[q][r, pl.ds(j * L, L)] * ps

    def issue_scatter(q):
        pltpu.async_copy(rows[q], acc_sh.at[sdix[q]], ssem.at[q], add=True)

        @pl.when(c == 0)
        def _():
            pltpu.async_copy(denr[q], den_sh.at[sdix[q]], dsem.at[q], add=True)

    def wait_scatter(q):
        pltpu.make_async_copy(rows[q], acc_sh.at[sdix[q]], ssem.at[q]).wait()

        @pl.when(c == 0)
        def _():
            pltpu.make_async_copy(denr[q], den_sh.at[sdix[q]], dsem.at[q]).wait()

    issue_idx(0, 0)
    issue_idx(1, 1)
    issue_idx(2, 2)
    wait_idx(0, 0)
    issue_rows(0)
    wait_idx(1, 1)
    issue_rows(1)

    def step(v, q):
        q1 = (q + 1) % NB
        q2 = (q + 2) % NB
        q3 = (q + 3) % NB

        @pl.when((v >= 2) & (v - 2 < nct))
        def _():
            wait_scatter(q2)

        @pl.when(v + 3 < nct)
        def _():
            issue_idx(v + 3, q3)

        @pl.when(v + 2 < nct)
        def _():
            wait_idx(v + 2, q2)
            issue_rows(q2)

        @pl.when(v < nct)
        def _():
            wait_rows(q)
            compute(q)
            issue_scatter(q)

    def quad(i, carry):
        for k in range(NB):
            step(NB * i + k, k)
        return carry

    niter = (full + 1 + 2 + NB - 1) // NB
    lax.fori_loop(0, niter, quad, 0)

    plsc.subcore_barrier()
    for q in range(nz):
        sl = pl.ds(base + q * zr, zr)
        pltpu.sync_copy(acc_sh.at[sl], acc_hbm.at[c, sl])

    @pl.when(c == 0)
    def _():
        for q in range(nz):
            sl = pl.ds(base + q * zr, zr)
            pltpu.sync_copy(den_sh.at[sl], den_hbm.at[sl])


def kernel(x, edge_index, W_gat, att_src, att_dst, bias_gat, W_proj, b_proj):
    n, d = x.shape
    e = edge_index.shape[1]
    bn = 1000
    grid = n // bn

    att2 = jnp.stack([att_src, att_dst], axis=1)

    hp, alphas, mx = pl.pallas_call(
        _pre_body,
        grid=(grid,),
        in_specs=[
            pl.BlockSpec((bn, d), lambda i: (i, 0)),
            pl.BlockSpec((d, d), lambda i: (0, 0)),
            pl.BlockSpec((d, 2), lambda i: (0, 0)),
        ],
        out_specs=[
            pl.BlockSpec((NC, bn, DH), lambda i: (0, i, 0)),
            pl.BlockSpec((bn, 2), lambda i: (i, 0)),
            pl.BlockSpec((1, 2), lambda i: (0, 0)),
        ],
        out_shape=[
            jax.ShapeDtypeStruct((NC, n, DH), jnp.float32),
            jax.ShapeDtypeStruct((n, 2), jnp.float32),
            jax.ShapeDtypeStruct((1, 2), jnp.float32),
        ],
    )(x, W_gat, att2)

    a_src = alphas[:, 0]
    a_dst = alphas[:, 1]
    shift = jnp.full((L,), jnp.maximum(mx[0, 0] + mx[0, 1], 0.0), jnp.float32)

    mesh = plsc.VectorSubcoreMesh(
        core_axis_name="c", subcore_axis_name="s", num_cores=NC, num_subcores=NS
    )
    edge_kernel = functools.partial(
        pl.kernel,
        out_type=(
            jax.ShapeDtypeStruct((NC, n, DH), jnp.float32),
            jax.ShapeDtypeStruct((n, L), jnp.float32),
        ),
        mesh=mesh,
        compiler_params=pltpu.CompilerParams(
            use_tc_tiling_on_sc=False, needs_layout_passes=False
        ),
        scratch_types=[
            pltpu.VMEM((n,), jnp.float32),
            pltpu.VMEM((n,), jnp.float32),
        ] + [pltpu.VMEM((2, C), jnp.int32) for _ in range(NB)]
          + [pltpu.VMEM((C, DH), jnp.float32) for _ in range(NB)]
          + [pltpu.VMEM((C, L), jnp.float32) for _ in range(NB)]
          + [pltpu.VMEM((C,), jnp.int32) for _ in range(NB)]
          + [
            pltpu.VMEM((C + L,), jnp.float32),
            pltpu.VMEM((L,), jnp.float32),
            pltpu.VMEM_SHARED((n, DH), jnp.float32),
            pltpu.VMEM_SHARED((n, L), jnp.float32),
            pltpu.SemaphoreType.DMA((NB,)),
            pltpu.SemaphoreType.DMA((NB,)),
            pltpu.SemaphoreType.DMA((NB,)),
            pltpu.SemaphoreType.DMA((NB,)),
        ],
    )(functools.partial(_edge_body, n_nodes=n, n_edges=e))

    acc, den = edge_kernel(hp, edge_index, a_src, a_dst, shift)

    out = pl.pallas_call(
        _post_body,
        grid=(grid,),
        in_specs=[
            pl.BlockSpec((NC, bn, DH), lambda i: (0, i, 0)),
            pl.BlockSpec((bn, L), lambda i: (i, 0)),
            pl.BlockSpec((1, d), lambda i: (0, 0)),
            pl.BlockSpec((d, d), lambda i: (0, 0)),
            pl.BlockSpec((1, d), lambda i: (0, 0)),
        ],
        out_specs=pl.BlockSpec((bn, d), lambda i: (i, 0)),
        out_shape=jax.ShapeDtypeStruct((n, d), jnp.float32),
    )(acc, den, bias_gat[None, :], W_proj, b_proj[None, :])

    return out


# speedup vs baseline: 2.1247x; 1.0376x over previous
import functools

import jax
import jax.numpy as jnp
from jax import lax
from jax.experimental import pallas as pl
from jax.experimental.pallas import tpu as pltpu
from jax.experimental.pallas import tpu_sc as plsc

NC = 2
NS = 16
L = 16
C = 128
DH = 64
NB = 4


def _pre_body(x_ref, w_ref, atts_ref, attd_ref, hp_ref, asrc_ref, adst_ref,
              shift_ref, mx_ref):
    i = pl.program_id(0)
    ng = pl.num_programs(0)
    h = jnp.dot(x_ref[...], w_ref[...], preferred_element_type=jnp.float32)
    att2 = jnp.concatenate([atts_ref[...], attd_ref[...]], axis=0)
    ab = lax.dot_general(h, att2, (((1,), (1,)), ((), ())),
                         preferred_element_type=jnp.float32)
    hp_ref[0, :, :] = h[:, :DH]
    hp_ref[1, :, :] = h[:, DH:]
    asrc_ref[...] = ab[:, 0:1]
    adst_ref[...] = ab[:, 1:2]
    bm = jnp.max(ab, axis=0, keepdims=True)

    @pl.when(i == 0)
    def _():
        mx_ref[...] = bm

    @pl.when(i != 0)
    def _():
        mx_ref[...] = jnp.maximum(mx_ref[...], bm)

    @pl.when(i == ng - 1)
    def _():
        m = jnp.maximum(mx_ref[0, 0] + mx_ref[0, 1], 0.0)
        shift_ref[...] = jnp.full((1, L), m, jnp.float32)


def _post_body(acc_ref, den_ref, bias_ref, wp_ref, bp_ref, out_ref):
    den = jnp.sum(den_ref[0] + den_ref[1], axis=1, keepdims=True) + 1e-16
    g0 = jnp.maximum(acc_ref[0] / den + bias_ref[:, :DH], 0.0)
    g1 = jnp.maximum(acc_ref[1] / den + bias_ref[:, DH:], 0.0)
    o = (jnp.dot(g0, wp_ref[:DH, :], preferred_element_type=jnp.float32)
         + jnp.dot(g1, wp_ref[DH:, :], preferred_element_type=jnp.float32)
         + bp_ref[...])
    out_ref[...] = jnp.maximum(o, 0.0)


def _edge_body(hp_hbm, edge_hbm, asrc_hbm, adst_hbm, shift_hbm,
               acc_hbm, den_hbm,
               asrc_v, adst_v, eidx0, eidx1, eidx2, eidx3,
               rows0, rows1, rows2, rows3, denr0, denr1, denr2, denr3,
               sdix0, sdix1, sdix2, sdix3,
               pbuf_v, shift_v, acc_sh, den_sh, isem, rsem, ssem, dsem,
               n_nodes, n_edges):
    c = lax.axis_index("c")
    s = lax.axis_index("s")
    eidx = (eidx0, eidx1, eidx2, eidx3)
    rows = (rows0, rows1, rows2, rows3)
    denr = (denr0, denr1, denr2, denr3)
    sdix = (sdix0, sdix1, sdix2, sdix3)
    rows_per_tile = n_nodes // NS
    zr = 125
    nz = rows_per_tile // zr

    pltpu.sync_copy(asrc_hbm, asrc_v)
    pltpu.sync_copy(adst_hbm, adst_v)
    pltpu.sync_copy(shift_hbm, shift_v)

    zeros16 = jnp.zeros((L,), jnp.float32)

    def zrow(r, carry):
        for j in range(DH // L):
            rows0[r, pl.ds(j * L, L)] = zeros16
        for q in range(NB):
            denr[q][r, :] = zeros16
        return carry

    lax.fori_loop(0, C, zrow, 0)
    base = s * rows_per_tile
    for q in range(nz):
        pltpu.sync_copy(rows0.at[pl.ds(0, zr)], acc_sh.at[pl.ds(base + q * zr, zr)])
        pltpu.sync_copy(denr0.at[pl.ds(0, zr)], den_sh.at[pl.ds(base + q * zr, zr)])

    plsc.subcore_barrier()

    shift = shift_v[0, :]
    iota16 = lax.iota(jnp.int32, L)
    zeros16i = jnp.zeros((L,), jnp.int32)
    nchunks = n_edges // C
    full = nchunks // NS
    rem = nchunks - full * NS
    nct = full + jnp.where(s < rem, 1, 0)

    def cid_of(u):
        return jnp.where(u < full, s + NS * u, NS * full + s)

    def issue_idx(u, q):
        e0 = cid_of(u) * C
        pltpu.async_copy(edge_hbm.at[:, pl.ds(e0, C)], eidx[q], isem.at[q])

    def wait_idx(u, q):
        e0 = cid_of(u) * C
        pltpu.make_async_copy(edge_hbm.at[:, pl.ds(e0, C)], eidx[q], isem.at[q]).wait()

    def issue_rows(q):
        pltpu.async_copy(hp_hbm.at[c].at[eidx[q].at[0]], rows[q], rsem.at[q])

    def wait_rows(q):
        pltpu.make_async_copy(hp_hbm.at[c].at[eidx[q].at[0]], rows[q], rsem.at[q]).wait()

    def compute(q, par):
        def grp_body(g, carry):
            si = eidx[q][0, pl.ds(g * L, L)]
            di = eidx[q][1, pl.ds(g * L, L)]
            sdix[q][pl.ds(g * L, L)] = di
            a = plsc.load_gather(asrc_v, [si]) + plsc.load_gather(adst_v, [di])
            a = jnp.where(a >= 0, a, 0.2 * a)
            p = jnp.exp(a - shift)
            pbuf_v[pl.ds(g * L, L)] = p

            @pl.when(c == par)
            def _():
                plsc.store_scatter(denr[q], [g * L + iota16, zeros16i], p)

            return carry

        lax.fori_loop(0, C // L, grp_body, 0)

        @plsc.parallel_loop(0, C, 1, unroll=8)
        def _(r):
            ps = pbuf_v[pl.ds(r, L)][0]
            for j in range(DH // L):
                rows[q][r, pl.ds(j * L, L)] = rows[q][r, pl.ds(j * L, L)] * ps

    def issue_scatter(q, par):
        pltpu.async_copy(rows[q], acc_sh.at[sdix[q]], ssem.at[q], add=True)

        @pl.when(c == par)
        def _():
            pltpu.async_copy(denr[q], den_sh.at[sdix[q]], dsem.at[q], add=True)

    def wait_scatter(q, par):
        pltpu.make_async_copy(rows[q], acc_sh.at[sdix[q]], ssem.at[q]).wait()

        @pl.when(c == par)
        def _():
            pltpu.make_async_copy(denr[q], den_sh.at[sdix[q]], dsem.at[q]).wait()

    issue_idx(0, 0)
    issue_idx(1, 1)
    issue_idx(2, 2)
    wait_idx(0, 0)
    issue_rows(0)
    wait_idx(1, 1)
    issue_rows(1)

    def step(v, q):
        q1 = (q + 1) % NB
        q2 = (q + 2) % NB
        q3 = (q + 3) % NB

        par = q % 2

        @pl.when((v >= 2) & (v - 2 < nct))
        def _():
            wait_scatter(q2, par)

        @pl.when(v + 3 < nct)
        def _():
            issue_idx(v + 3, q3)

        @pl.when(v + 2 < nct)
        def _():
            wait_idx(v + 2, q2)
            issue_rows(q2)

        @pl.when(v < nct)
        def _():
            wait_rows(q)
            compute(q, par)
            issue_scatter(q, par)

    def quad(i, carry):
        for k in range(NB):
            step(NB * i + k, k)
        return carry

    niter = (full + 1 + 2 + NB - 1) // NB
    lax.fori_loop(0, niter, quad, 0)

    plsc.subcore_barrier()
    for q in range(nz):
        sl = pl.ds(base + q * zr, zr)
        pltpu.sync_copy(acc_sh.at[sl], acc_hbm.at[c, sl])
        pltpu.sync_copy(den_sh.at[sl], den_hbm.at[c, sl])


def kernel(x, edge_index, W_gat, att_src, att_dst, bias_gat, W_proj, b_proj):
    n, d = x.shape
    e = edge_index.shape[1]
    bn = 1000
    grid = n // bn

    hp, a_src, a_dst, shift = pl.pallas_call(
        _pre_body,
        grid=(grid,),
        in_specs=[
            pl.BlockSpec((bn, d), lambda i: (i, 0)),
            pl.BlockSpec((d, d), lambda i: (0, 0)),
            pl.BlockSpec((1, d), lambda i: (0, 0)),
            pl.BlockSpec((1, d), lambda i: (0, 0)),
        ],
        out_specs=[
            pl.BlockSpec((NC, bn, DH), lambda i: (0, i, 0)),
            pl.BlockSpec((bn, 1), lambda i: (i, 0)),
            pl.BlockSpec((bn, 1), lambda i: (i, 0)),
            pl.BlockSpec((1, L), lambda i: (0, 0)),
        ],
        out_shape=[
            jax.ShapeDtypeStruct((NC, n, DH), jnp.float32),
            jax.ShapeDtypeStruct((n, 1), jnp.float32),
            jax.ShapeDtypeStruct((n, 1), jnp.float32),
            jax.ShapeDtypeStruct((1, L), jnp.float32),
        ],
        scratch_shapes=[pltpu.VMEM((1, 2), jnp.float32)],
    )(x, W_gat, att_src[None, :], att_dst[None, :])

    mesh = plsc.VectorSubcoreMesh(
        core_axis_name="c", subcore_axis_name="s", num_cores=NC, num_subcores=NS
    )
    edge_kernel = functools.partial(
        pl.kernel,
        out_type=(
            jax.ShapeDtypeStruct((NC, n, DH), jnp.float32),
            jax.ShapeDtypeStruct((NC, n, L), jnp.float32),
        ),
        mesh=mesh,
        compiler_params=pltpu.CompilerParams(
            use_tc_tiling_on_sc=False, needs_layout_passes=False
        ),
        scratch_types=[
            pltpu.VMEM((n,), jnp.float32),
            pltpu.VMEM((n,), jnp.float32),
        ] + [pltpu.VMEM((2, C), jnp.int32) for _ in range(NB)]
          + [pltpu.VMEM((C, DH), jnp.float32) for _ in range(NB)]
          + [pltpu.VMEM((C, L), jnp.float32) for _ in range(NB)]
          + [pltpu.VMEM((C,), jnp.int32) for _ in range(NB)]
          + [
            pltpu.VMEM((C + L,), jnp.float32),
            pltpu.VMEM((1, L), jnp.float32),
            pltpu.VMEM_SHARED((n, DH), jnp.float32),
            pltpu.VMEM_SHARED((n, L), jnp.float32),
            pltpu.SemaphoreType.DMA((NB,)),
            pltpu.SemaphoreType.DMA((NB,)),
            pltpu.SemaphoreType.DMA((NB,)),
            pltpu.SemaphoreType.DMA((NB,)),
        ],
    )(functools.partial(_edge_body, n_nodes=n, n_edges=e))

    acc, den = edge_kernel(hp, edge_index, a_src.reshape(n), a_dst.reshape(n), shift)

    out = pl.pallas_call(
        _post_body,
        grid=(grid,),
        in_specs=[
            pl.BlockSpec((NC, bn, DH), lambda i: (0, i, 0)),
            pl.BlockSpec((NC, bn, L), lambda i: (0, i, 0)),
            pl.BlockSpec((1, d), lambda i: (0, 0)),
            pl.BlockSpec((d, d), lambda i: (0, 0)),
            pl.BlockSpec((1, d), lambda i: (0, 0)),
        ],
        out_specs=pl.BlockSpec((bn, d), lambda i: (i, 0)),
        out_shape=jax.ShapeDtypeStruct((n, d), jnp.float32),
    )(acc, den, bias_gat[None, :], W_proj, b_proj[None, :])

    return out
